# baseline (speedup 1.0000x reference)
import jax
import jax.numpy as jnp
from jax import lax
from jax.experimental import pallas as pl
from jax.experimental.pallas import tpu as pltpu
from jax.experimental.pallas import tpu_sc as plsc

BATCH = 16384
NC = 2
NS = 16
NW = NC * NS
B_PER_W = BATCH // NW
CHUNK = 128
NCHUNK = B_PER_W // CHUNK
LANES = 16


def _irt_body(stu_idx_hbm, exer_idx_hbm, ws_hbm, wk_hbm, we_hbm, out_hbm,
              sidx_v, eidx_v, s_v, k_v, e_v, out_v, sem):
    wid = lax.axis_index("s") * NC + lax.axis_index("c")
    base = wid * B_PER_W

    pltpu.sync_copy(stu_idx_hbm.at[wid], sidx_v)
    pltpu.sync_copy(exer_idx_hbm.at[wid], eidx_v)

    copies = [
        pltpu.async_copy(ws_hbm.at[sidx_v], s_v, sem),
        pltpu.async_copy(wk_hbm.at[eidx_v], k_v, sem),
        pltpu.async_copy(we_hbm.at[eidx_v], e_v, sem),
    ]
    for c in copies:
        c.wait()

    one = jnp.full((LANES,), 1.0, dtype=jnp.float32)
    for i in range(B_PER_W // LANES):
        sl = pl.ds(i * LANES, LANES)
        s = s_v[sl]
        k = k_v[sl]
        e = e_v[sl]
        s_sig = one / (one + jnp.exp(-s))
        k_sig = one / (one + jnp.exp(-k))
        e_sig = one / (one + jnp.exp(-e))
        z = e_sig * 1.7 * (s_sig - k_sig)
        out_v[sl] = one / (one + jnp.exp(-z))

    pltpu.sync_copy(out_v, out_hbm.at[pl.ds(base, B_PER_W)])


@jax.jit
def _irt_sc(stu_idx, exer_idx, ws, wk, we):
    mesh = plsc.VectorSubcoreMesh(core_axis_name="c", subcore_axis_name="s")
    return pl.kernel(
        _irt_body,
        mesh=mesh,
        out_type=jax.ShapeDtypeStruct((BATCH,), jnp.float32),
        scratch_types=[
            pltpu.VMEM((B_PER_W,), jnp.int32),
            pltpu.VMEM((B_PER_W,), jnp.int32),
            pltpu.VMEM((B_PER_W,), jnp.float32),
            pltpu.VMEM((B_PER_W,), jnp.float32),
            pltpu.VMEM((B_PER_W,), jnp.float32),
            pltpu.VMEM((B_PER_W,), jnp.float32),
            pltpu.SemaphoreType.DMA,
        ],
    )(stu_idx, exer_idx, ws, wk, we)


def kernel(stu_id, exer_id, W_student, W_k_difficulty, W_e_discrimination):
    stu_idx = stu_id.astype(jnp.int32).reshape(NW, B_PER_W)
    exer_idx = exer_id.astype(jnp.int32).reshape(NW, B_PER_W)
    ws = W_student.reshape(-1)
    wk = W_k_difficulty.reshape(-1)
    we = W_e_discrimination.reshape(-1)
    return _irt_sc(stu_idx, exer_idx, ws, wk, we)


# speedup vs baseline: 1.0575x; 1.0575x over previous
import jax
import jax.numpy as jnp
from jax import lax
from jax.experimental import pallas as pl
from jax.experimental.pallas import tpu as pltpu
from jax.experimental.pallas import tpu_sc as plsc

BATCH = 16384
NC = 2
NS = 16
NW = NC * NS
B_PER_W = BATCH // NW
CHUNK = 128
NCHUNK = B_PER_W // CHUNK
LANES = 16


def _irt_body(stu_idx_hbm, exer_idx_hbm, ws_hbm, wk_hbm, we_hbm, out_hbm,
              sidx_v, eidx_v, s_v, k_v, e_v, out_v, isem, *sems):
    wid = lax.axis_index("s") * NC + lax.axis_index("c")
    base = wid * B_PER_W

    ic0 = pltpu.async_copy(stu_idx_hbm.at[wid], sidx_v, isem)
    ic1 = pltpu.async_copy(exer_idx_hbm.at[wid], eidx_v, isem)
    ic0.wait()
    ic1.wait()

    copies = []
    for j in range(NCHUNK):
        copies.append((
            pltpu.async_copy(ws_hbm.at[sidx_v.at[j]], s_v.at[j], sems[j]),
            pltpu.async_copy(wk_hbm.at[eidx_v.at[j]], k_v.at[j], sems[j]),
            pltpu.async_copy(we_hbm.at[eidx_v.at[j]], e_v.at[j], sems[j]),
        ))

    one = jnp.full((LANES,), 1.0, dtype=jnp.float32)
    for j in range(NCHUNK):
        for c in copies[j]:
            c.wait()
        for i in range(CHUNK // LANES):
            sl = pl.ds(i * LANES, LANES)
            es = jnp.exp(-s_v[j, sl])
            ek = jnp.exp(-k_v[j, sl])
            ee = jnp.exp(-e_v[j, sl])
            z = (1.7 * (ek - es)) / ((one + ee) * (one + es) * (one + ek))
            out_v[pl.ds(j * CHUNK + i * LANES, LANES)] = one / (one + jnp.exp(-z))

    pltpu.sync_copy(out_v, out_hbm.at[pl.ds(base, B_PER_W)])


@jax.jit
def _irt_sc(stu_idx, exer_idx, ws, wk, we):
    mesh = plsc.VectorSubcoreMesh(core_axis_name="c", subcore_axis_name="s")
    return pl.kernel(
        _irt_body,
        mesh=mesh,
        out_type=jax.ShapeDtypeStruct((BATCH,), jnp.float32),
        scratch_types=[
            pltpu.VMEM((NCHUNK, CHUNK), jnp.int32),
            pltpu.VMEM((NCHUNK, CHUNK), jnp.int32),
            pltpu.VMEM((NCHUNK, CHUNK), jnp.float32),
            pltpu.VMEM((NCHUNK, CHUNK), jnp.float32),
            pltpu.VMEM((NCHUNK, CHUNK), jnp.float32),
            pltpu.VMEM((B_PER_W,), jnp.float32),
            pltpu.SemaphoreType.DMA,
        ] + [pltpu.SemaphoreType.DMA] * NCHUNK,
    )(stu_idx, exer_idx, ws, wk, we)


def kernel(stu_id, exer_id, W_student, W_k_difficulty, W_e_discrimination):
    stu_idx = stu_id.astype(jnp.int32).reshape(NW, NCHUNK, CHUNK)
    exer_idx = exer_id.astype(jnp.int32).reshape(NW, NCHUNK, CHUNK)
    ws = W_student.reshape(-1)
    wk = W_k_difficulty.reshape(-1)
    we = W_e_discrimination.reshape(-1)
    return _irt_sc(stu_idx, exer_idx, ws, wk, we)


# speedup vs baseline: 1.0624x; 1.0047x over previous
import jax
import jax.numpy as jnp
from jax import lax
from jax.experimental import pallas as pl
from jax.experimental.pallas import tpu as pltpu
from jax.experimental.pallas import tpu_sc as plsc

BATCH = 16384
NC = 2
NS = 16
NW = NC * NS
B_PER_W = BATCH // NW
CHUNK = 128
NCHUNK = B_PER_W // CHUNK
LANES = 16


def _irt_body(stu_idx_hbm, exer_idx_hbm, ws_hbm, wk_hbm, we_hbm, out_hbm,
              sidx_v, eidx_v, s_v, k_v, e_v, out_v, osem, *sems):
    wid = lax.axis_index("s") * NC + lax.axis_index("c")
    base = wid * B_PER_W

    idx_copies = [
        (
            pltpu.async_copy(stu_idx_hbm.at[wid, j], sidx_v.at[j], sems[j]),
            pltpu.async_copy(exer_idx_hbm.at[wid, j], eidx_v.at[j], sems[j]),
        )
        for j in range(NCHUNK)
    ]

    gathers = []
    for j in range(NCHUNK):
        for c in idx_copies[j]:
            c.wait()
        gathers.append((
            pltpu.async_copy(ws_hbm.at[sidx_v.at[j]], s_v.at[j], sems[j]),
            pltpu.async_copy(wk_hbm.at[eidx_v.at[j]], k_v.at[j], sems[j]),
            pltpu.async_copy(we_hbm.at[eidx_v.at[j]], e_v.at[j], sems[j]),
        ))

    one = jnp.full((LANES,), 1.0, dtype=jnp.float32)
    out_copies = []
    for j in range(NCHUNK):
        for c in gathers[j]:
            c.wait()
        for i in range(CHUNK // LANES):
            sl = pl.ds(i * LANES, LANES)
            es = jnp.exp(-s_v[j, sl])
            ek = jnp.exp(-k_v[j, sl])
            ee = jnp.exp(-e_v[j, sl])
            z = (1.7 * (ek - es)) / ((one + ee) * (one + es) * (one + ek))
            out_v[pl.ds(j * CHUNK + i * LANES, LANES)] = one / (one + jnp.exp(-z))
        out_copies.append(pltpu.async_copy(
            out_v.at[pl.ds(j * CHUNK, CHUNK)],
            out_hbm.at[pl.ds(base + j * CHUNK, CHUNK)], osem))
    for c in out_copies:
        c.wait()


@jax.jit
def _irt_sc(stu_idx, exer_idx, ws, wk, we):
    mesh = plsc.VectorSubcoreMesh(core_axis_name="c", subcore_axis_name="s")
    return pl.kernel(
        _irt_body,
        mesh=mesh,
        out_type=jax.ShapeDtypeStruct((BATCH,), jnp.float32),
        scratch_types=[
            pltpu.VMEM((NCHUNK, CHUNK), jnp.int32),
            pltpu.VMEM((NCHUNK, CHUNK), jnp.int32),
            pltpu.VMEM((NCHUNK, CHUNK), jnp.float32),
            pltpu.VMEM((NCHUNK, CHUNK), jnp.float32),
            pltpu.VMEM((NCHUNK, CHUNK), jnp.float32),
            pltpu.VMEM((B_PER_W,), jnp.float32),
            pltpu.SemaphoreType.DMA,
        ] + [pltpu.SemaphoreType.DMA] * NCHUNK,
    )(stu_idx, exer_idx, ws, wk, we)


def kernel(stu_id, exer_id, W_student, W_k_difficulty, W_e_discrimination):
    stu_idx = stu_id.astype(jnp.int32).reshape(NW, NCHUNK, CHUNK)
    exer_idx = exer_id.astype(jnp.int32).reshape(NW, NCHUNK, CHUNK)
    ws = W_student.reshape(-1)
    wk = W_k_difficulty.reshape(-1)
    we = W_e_discrimination.reshape(-1)
    return _irt_sc(stu_idx, exer_idx, ws, wk, we)
